# baseline (speedup 1.0000x reference)
import functools
import math

import jax
import jax.numpy as jnp
from jax.experimental import pallas as pl
from jax.experimental.pallas import tpu as pltpu

_B, _S, _DIM = 1, 2048, 2048
_H, _DH = 16, 128
_RQ = 512
_G, _RO = 4, 128
_EPS = 1e-6
_BQ = 256
_LOG2E = 1.4426950408889634


def _dot(a, b, dims):
    return jax.lax.dot_general(a, b, (dims, ((), ())),
                               preferred_element_type=jnp.float32)


def _proj_body(x_ref, wqd_ref, qln_ref, wqu_ref, wkv_ref, kvln_ref,
               q_ref, kv_ref):
    xb = x_ref[...]
    kvh = _dot(xb, wkv_ref[...], ((1,), (1,)))
    var = jnp.mean(kvh * kvh, axis=-1, keepdims=True)
    kvn = kvh * jax.lax.rsqrt(var + _EPS) * kvln_ref[...]
    kv_ref[...] = kvn.astype(jnp.bfloat16)

    qh = _dot(xb, wqd_ref[...], ((1,), (1,)))
    qvar = jnp.mean(qh * qh, axis=-1, keepdims=True)
    qn = (qh * jax.lax.rsqrt(qvar + _EPS) * qln_ref[...]).astype(jnp.bfloat16)
    qb = _dot(qn, wqu_ref[...], ((1,), (1,)))
    q_ref[...] = (qb * (_LOG2E / math.sqrt(_DH))).astype(jnp.bfloat16)


def _attn_body(q_ref, kv_ref, sink_ref, wod_ref, wou_ref, o_ref):
    i = pl.program_id(0)
    qsb = q_ref[...]
    kv_all = kv_ref[...]
    rows = i * _BQ + jax.lax.broadcasted_iota(jnp.int32, (_BQ, _S), 0)
    cols = jax.lax.broadcasted_iota(jnp.int32, (_BQ, _S), 1)
    mask = cols <= rows
    esink = jax.lax.exp2(sink_ref[...] * _LOG2E)

    parts = []
    for h in range(_H):
        q_h = qsb[:, h * _DH:(h + 1) * _DH]
        e = jnp.where(mask,
                      jax.lax.exp2(_dot(q_h, kv_all, ((1,), (1,)))), 0.0)
        denom = jnp.sum(e, axis=-1, keepdims=True) + esink[0, h]
        acc = _dot(e.astype(jnp.bfloat16), kv_all, ((1,), (0,)))
        parts.append(acc / denom)
    att = jnp.concatenate(parts, axis=1)

    z_parts = []
    for g in range(_G):
        og = att[:, g * (_H // _G) * _DH:(g + 1) * (_H // _G) * _DH]
        wdg = wod_ref[g * _RO:(g + 1) * _RO, :]
        z_parts.append(_dot(og.astype(jnp.bfloat16), wdg, ((1,), (1,))))
    z = jnp.concatenate(z_parts, axis=1).astype(jnp.bfloat16)
    o_ref[...] = _dot(z, wou_ref[...], ((1,), (1,)))


@functools.partial(jax.jit, static_argnames=())
def kernel(x, wq_down, q_ln, wq_up, wkv, kv_ln, attn_sink, wo_down, wo_up):
    xs = x.reshape(_S, _DIM).astype(jnp.bfloat16)
    fullp = lambda shape: pl.BlockSpec(shape, lambda i: (0, 0))
    qs, kv = pl.pallas_call(
        _proj_body,
        grid=(_S // _BQ,),
        in_specs=[
            pl.BlockSpec((_BQ, _DIM), lambda i: (i, 0)),
            fullp((_RQ, _DIM)),
            fullp((1, _RQ)),
            fullp((_H * _DH, _RQ)),
            fullp((_DH, _DIM)),
            fullp((1, _DH)),
        ],
        out_specs=[pl.BlockSpec((_BQ, _H * _DH), lambda i: (i, 0)),
                   pl.BlockSpec((_BQ, _DH), lambda i: (i, 0))],
        out_shape=[jax.ShapeDtypeStruct((_S, _H * _DH), jnp.bfloat16),
                   jax.ShapeDtypeStruct((_S, _DH), jnp.bfloat16)],
        compiler_params=pltpu.CompilerParams(
            dimension_semantics=("parallel",)),
    )(
        xs,
        wq_down.astype(jnp.bfloat16),
        q_ln.reshape(1, _RQ),
        wq_up.astype(jnp.bfloat16),
        wkv.astype(jnp.bfloat16),
        kv_ln.reshape(1, _DH),
    )
    out = pl.pallas_call(
        _attn_body,
        grid=(_S // _BQ,),
        in_specs=[
            pl.BlockSpec((_BQ, _H * _DH), lambda i: (i, 0)),
            pl.BlockSpec((_S, _DH), lambda i: (0, 0)),
            fullp((1, _H)),
            fullp((_G * _RO, (_H * _DH) // _G)),
            fullp((_DIM, _G * _RO)),
        ],
        out_specs=pl.BlockSpec((_BQ, _DIM), lambda i: (i, 0)),
        out_shape=jax.ShapeDtypeStruct((_S, _DIM), jnp.float32),
        compiler_params=pltpu.CompilerParams(
            dimension_semantics=("parallel",)),
    )(
        qs,
        kv,
        attn_sink.reshape(1, _H),
        wo_down.astype(jnp.bfloat16),
        wo_up.astype(jnp.bfloat16),
    )
    return out.reshape(_B, _S, _DIM)


# speedup vs baseline: 1.2201x; 1.2201x over previous
import functools
import math

import jax
import jax.numpy as jnp
from jax.experimental import pallas as pl
from jax.experimental.pallas import tpu as pltpu

_B, _S, _DIM = 1, 2048, 2048
_H, _DH = 16, 128
_RQ = 512
_G, _RO = 4, 128
_EPS = 1e-6
_BQ = 256
_LOG2E = 1.4426950408889634


def _dot(a, b, dims):
    return jax.lax.dot_general(a, b, (dims, ((), ())),
                               preferred_element_type=jnp.float32)


def _body(x_ref, wqd_ref, qln_ref, wqu_ref, wkv_ref, kvln_ref, sink_ref,
          wod_ref, wou_ref, o_ref, kv_scr, q_scr, acc_scr, den_scr,
          wqd16, wqu16, wkv16, wod16, wou16):
    i = pl.program_id(0)
    j = pl.program_id(1)

    @pl.when(jnp.logical_and(i == 0, j == 0))
    def _wcast():
        wqd16[...] = wqd_ref[...].astype(jnp.bfloat16)
        wqu16[...] = wqu_ref[...].astype(jnp.bfloat16)
        wkv16[...] = wkv_ref[...].astype(jnp.bfloat16)
        wod16[...] = wod_ref[...].astype(jnp.bfloat16)
        wou16[...] = wou_ref[...].astype(jnp.bfloat16)

    @pl.when(j == 0)
    def _proj():
        xb = x_ref[...].astype(jnp.bfloat16)
        kvh = _dot(xb, wkv16[...], ((1,), (1,)))
        var = jnp.mean(kvh * kvh, axis=-1, keepdims=True)
        kvn = kvh * jax.lax.rsqrt(var + _EPS) * kvln_ref[...]
        kv_scr[pl.ds(i * _BQ, _BQ), :] = kvn.astype(jnp.bfloat16)
        qh = _dot(xb, wqd16[...], ((1,), (1,)))
        qvar = jnp.mean(qh * qh, axis=-1, keepdims=True)
        qn = (qh * jax.lax.rsqrt(qvar + _EPS) * qln_ref[...]
              ).astype(jnp.bfloat16)
        qb = _dot(qn, wqu16[...], ((1,), (1,)))
        q_scr[...] = (qb * (_LOG2E / math.sqrt(_DH))).astype(jnp.bfloat16)
        acc_scr[...] = jnp.zeros((_BQ, _H * _DH), jnp.float32)
        esink = jax.lax.exp2(sink_ref[...] * _LOG2E)
        den_scr[...] = jnp.broadcast_to(esink, (_BQ, _H))

    @pl.when(j <= i)
    def _attend():
        kv_j = kv_scr[pl.ds(j * _BQ, _BQ), :]
        qsb = q_scr[...]
        hg = 4
        mq = hg * _BQ
        r_loc = jax.lax.broadcasted_iota(jnp.int32, (mq, _BQ), 0)
        c_loc = jax.lax.broadcasted_iota(jnp.int32, (mq, _BQ), 1)
        mask = j * _BQ + c_loc <= i * _BQ + jax.lax.rem(r_loc, _BQ)
        accs, dens = [], []
        for g in range(_H // hg):
            q_g = jnp.concatenate(
                [qsb[:, (g * hg + hh) * _DH:(g * hg + hh + 1) * _DH]
                 for hh in range(hg)], axis=0)
            e = jnp.where(mask,
                          jax.lax.exp2(_dot(q_g, kv_j, ((1,), (1,)))), 0.0)
            den_g = jnp.sum(e, axis=-1, keepdims=True)
            pv_g = _dot(e.astype(jnp.bfloat16), kv_j, ((1,), (0,)))
            accs.append(pv_g)
            dens.append(den_g)
        acc_upd = jnp.concatenate(
            [accs[hh // hg][(hh % hg) * _BQ:(hh % hg + 1) * _BQ, :]
             for hh in range(_H)], axis=1)
        den_upd = jnp.concatenate(
            [dens[hh // hg][(hh % hg) * _BQ:(hh % hg + 1) * _BQ, :]
             for hh in range(_H)], axis=1)
        acc_scr[...] += acc_upd
        den_scr[...] += den_upd

    @pl.when(j == i)
    def _finalize():
        acc = acc_scr[...]
        den = den_scr[...]
        att_parts = [acc[:, h * _DH:(h + 1) * _DH] / den[:, h:h + 1]
                     for h in range(_H)]
        att = jnp.concatenate(att_parts, axis=1)
        z_parts = []
        for g in range(_G):
            og = att[:, g * (_H // _G) * _DH:(g + 1) * (_H // _G) * _DH]
            wdg = wod16[g * _RO:(g + 1) * _RO, :]
            z_parts.append(_dot(og.astype(jnp.bfloat16), wdg, ((1,), (1,))))
        z = jnp.concatenate(z_parts, axis=1).astype(jnp.bfloat16)
        o_ref[...] = _dot(z, wou16[...], ((1,), (1,)))


@functools.partial(jax.jit, static_argnames=())
def kernel(x, wq_down, q_ln, wq_up, wkv, kv_ln, attn_sink, wo_down, wo_up):
    xs = x.reshape(_S, _DIM)
    full = lambda shape: pl.BlockSpec(shape, lambda i, j: (0, 0))
    out = pl.pallas_call(
        _body,
        grid=(_S // _BQ, _S // _BQ),
        in_specs=[
            pl.BlockSpec((_BQ, _DIM), lambda i, j: (i, 0)),
            full((_RQ, _DIM)),
            full((1, _RQ)),
            full((_H * _DH, _RQ)),
            full((_DH, _DIM)),
            full((1, _DH)),
            full((1, _H)),
            full((_G * _RO, (_H * _DH) // _G)),
            full((_DIM, _G * _RO)),
        ],
        out_specs=pl.BlockSpec((_BQ, _DIM), lambda i, j: (i, 0)),
        out_shape=jax.ShapeDtypeStruct((_S, _DIM), jnp.float32),
        scratch_shapes=[pltpu.VMEM((_S, _DH), jnp.bfloat16),
                        pltpu.VMEM((_BQ, _H * _DH), jnp.bfloat16),
                        pltpu.VMEM((_BQ, _H * _DH), jnp.float32),
                        pltpu.VMEM((_BQ, _H), jnp.float32),
                        pltpu.VMEM((_RQ, _DIM), jnp.bfloat16),
                        pltpu.VMEM((_H * _DH, _RQ), jnp.bfloat16),
                        pltpu.VMEM((_DH, _DIM), jnp.bfloat16),
                        pltpu.VMEM((_G * _RO, (_H * _DH) // _G), jnp.bfloat16),
                        pltpu.VMEM((_DIM, _G * _RO), jnp.bfloat16)],
        compiler_params=pltpu.CompilerParams(
            dimension_semantics=("arbitrary", "arbitrary")),
    )(
        xs,
        wq_down,
        q_ln.reshape(1, _RQ),
        wq_up,
        wkv,
        kv_ln.reshape(1, _DH),
        attn_sink.reshape(1, _H),
        wo_down,
        wo_up,
    )
    return out.reshape(_B, _S, _DIM)


# speedup vs baseline: 1.2228x; 1.0022x over previous
import functools
import math

import jax
import jax.numpy as jnp
from jax.experimental import pallas as pl
from jax.experimental.pallas import tpu as pltpu

_B, _S, _DIM = 1, 2048, 2048
_H, _DH = 16, 128
_RQ = 512
_G, _RO = 4, 128
_EPS = 1e-6
_BQ = 256
_BK = 512
_MQ = _H * _BQ
_LOG2E = 1.4426950408889634


def _dot(a, b, dims):
    return jax.lax.dot_general(a, b, (dims, ((), ())),
                               preferred_element_type=jnp.float32)


def _body(x_ref, wqd_ref, qln_ref, wqu_ref, wkv_ref, kvln_ref, sink_ref,
          wod_ref, wou_ref, o_ref, kv_scr, q_scr, acc_scr, den_scr,
          wqd16, wqu16, wkv16, wod16, wou16):
    i = pl.program_id(0)
    j = pl.program_id(1)

    @pl.when(jnp.logical_and(i == 0, j == 0))
    def _init():
        wqd16[...] = wqd_ref[...].astype(jnp.bfloat16)
        wqu16[...] = wqu_ref[...].astype(jnp.bfloat16)
        wkv16[...] = wkv_ref[...].astype(jnp.bfloat16)
        wod16[...] = wod_ref[...].astype(jnp.bfloat16)
        wou16[...] = wou_ref[...].astype(jnp.bfloat16)
        kv_scr[...] = jnp.zeros((_S, _DH), jnp.bfloat16)

    @pl.when(j == 0)
    def _proj():
        xb = x_ref[...].astype(jnp.bfloat16)
        kvh = _dot(xb, wkv16[...], ((1,), (1,)))
        var = jnp.mean(kvh * kvh, axis=-1, keepdims=True)
        kvn = kvh * jax.lax.rsqrt(var + _EPS) * kvln_ref[...]
        kv_scr[pl.ds(i * _BQ, _BQ), :] = kvn.astype(jnp.bfloat16)
        qh = _dot(xb, wqd16[...], ((1,), (1,)))
        qvar = jnp.mean(qh * qh, axis=-1, keepdims=True)
        qn = (qh * jax.lax.rsqrt(qvar + _EPS) * qln_ref[...]
              ).astype(jnp.bfloat16)
        qb = _dot(qn, wqu16[...], ((1,), (1,)))
        qbs = (qb * (_LOG2E / math.sqrt(_DH))).astype(jnp.bfloat16)
        q_scr[...] = jnp.concatenate(
            [qbs[:, h * _DH:(h + 1) * _DH] for h in range(_H)], axis=0)
        acc_scr[...] = jnp.zeros((_MQ, _DH), jnp.float32)
        esink = jax.lax.exp2(sink_ref[...] * _LOG2E)
        den_scr[...] = jnp.concatenate(
            [jnp.broadcast_to(esink[0, h], (_BQ, 1)) for h in range(_H)],
            axis=0)

    @pl.when(2 * j <= i)
    def _attend():
        kv_j = kv_scr[pl.ds(j * _BK, _BK), :]
        q_stk = q_scr[...]
        r_loc = jax.lax.broadcasted_iota(jnp.int32, (_MQ, _BK), 0)
        c_loc = jax.lax.broadcasted_iota(jnp.int32, (_MQ, _BK), 1)
        mask = (j * _BK + c_loc
                <= i * _BQ + jax.lax.bitwise_and(r_loc, _BQ - 1))
        e = jnp.where(mask,
                      jax.lax.exp2(_dot(q_stk, kv_j, ((1,), (1,)))), 0.0)
        den_scr[...] += jnp.sum(e, axis=-1, keepdims=True)
        acc_scr[...] += _dot(e.astype(jnp.bfloat16), kv_j, ((1,), (0,)))

    @pl.when(j == jax.lax.div(i, 2))
    def _finalize():
        att_stk = acc_scr[...] / den_scr[...]
        att = jnp.concatenate(
            [att_stk[h * _BQ:(h + 1) * _BQ, :] for h in range(_H)], axis=1)
        z_parts = []
        for g in range(_G):
            og = att[:, g * (_H // _G) * _DH:(g + 1) * (_H // _G) * _DH]
            wdg = wod16[g * _RO:(g + 1) * _RO, :]
            z_parts.append(_dot(og.astype(jnp.bfloat16), wdg, ((1,), (1,))))
        z = jnp.concatenate(z_parts, axis=1).astype(jnp.bfloat16)
        o_ref[...] = _dot(z, wou16[...], ((1,), (1,)))


@functools.partial(jax.jit, static_argnames=())
def kernel(x, wq_down, q_ln, wq_up, wkv, kv_ln, attn_sink, wo_down, wo_up):
    xs = x.reshape(_S, _DIM)
    full = lambda shape: pl.BlockSpec(shape, lambda i, j: (0, 0))
    out = pl.pallas_call(
        _body,
        grid=(_S // _BQ, _S // _BK),
        in_specs=[
            pl.BlockSpec((_BQ, _DIM), lambda i, j: (i, 0)),
            full((_RQ, _DIM)),
            full((1, _RQ)),
            full((_H * _DH, _RQ)),
            full((_DH, _DIM)),
            full((1, _DH)),
            full((1, _H)),
            full((_G * _RO, (_H * _DH) // _G)),
            full((_DIM, _G * _RO)),
        ],
        out_specs=pl.BlockSpec((_BQ, _DIM), lambda i, j: (i, 0)),
        out_shape=jax.ShapeDtypeStruct((_S, _DIM), jnp.float32),
        scratch_shapes=[pltpu.VMEM((_S, _DH), jnp.bfloat16),
                        pltpu.VMEM((_MQ, _DH), jnp.bfloat16),
                        pltpu.VMEM((_MQ, _DH), jnp.float32),
                        pltpu.VMEM((_MQ, 1), jnp.float32),
                        pltpu.VMEM((_RQ, _DIM), jnp.bfloat16),
                        pltpu.VMEM((_H * _DH, _RQ), jnp.bfloat16),
                        pltpu.VMEM((_DH, _DIM), jnp.bfloat16),
                        pltpu.VMEM((_G * _RO, (_H * _DH) // _G), jnp.bfloat16),
                        pltpu.VMEM((_DIM, _G * _RO), jnp.bfloat16)],
        compiler_params=pltpu.CompilerParams(
            dimension_semantics=("arbitrary", "arbitrary")),
    )(
        xs,
        wq_down,
        q_ln.reshape(1, _RQ),
        wq_up,
        wkv,
        kv_ln.reshape(1, _DH),
        attn_sink.reshape(1, _H),
        wo_down,
        wo_up,
    )
    return out.reshape(_B, _S, _DIM)
